# baseline (speedup 1.0000x reference)
import functools
import jax
import jax.numpy as jnp
from jax import lax
from jax.experimental import pallas as pl
from jax.experimental.pallas import tpu as pltpu
from jax.experimental.pallas import tpu_sc as plsc

CHUNK = 128


@functools.cache
def _build(B, V, D):
    info = plsc.get_sparse_core_info()
    nw = info.num_cores * info.num_subcores
    b_per_w = B // nw
    n_chunks = b_per_w // CHUNK
    mesh = plsc.VectorSubcoreMesh(core_axis_name="c", subcore_axis_name="s")

    @functools.partial(
        pl.kernel,
        mesh=mesh,
        out_type=jax.ShapeDtypeStruct((B, D), jnp.float32),
        compiler_params=pltpu.CompilerParams(use_tc_tiling_on_sc=False),
        scratch_types=[
            pltpu.VMEM((b_per_w,), jnp.int32),
            pltpu.VMEM((b_per_w, D), jnp.float32),
            pltpu.SemaphoreType.DMA,
        ],
    )
    def k(idx_hbm, table_hbm, out_hbm, idx_v, rows_v, sem):
        wid = lax.axis_index("s") * info.num_cores + lax.axis_index("c")
        base = wid * b_per_w
        pltpu.sync_copy(idx_hbm.at[pl.ds(base, b_per_w)], idx_v)
        copies = []
        for j in range(n_chunks):
            copies.append(
                pltpu.async_copy(
                    table_hbm.at[idx_v.at[pl.ds(j * CHUNK, CHUNK)]],
                    rows_v.at[pl.ds(j * CHUNK, CHUNK)],
                    sem,
                )
            )
        for c in copies:
            c.wait()
        pltpu.sync_copy(rows_v, out_hbm.at[pl.ds(base, b_per_w)])

    return k


def kernel(nombre, table):
    B = nombre.shape[0]
    V, D = table.shape
    idx = nombre.astype(jnp.int32)
    return _build(B, V, D)(idx, table)


# speedup vs baseline: 1.2894x; 1.2894x over previous
import functools
import jax
import jax.numpy as jnp
from jax import lax
from jax.experimental import pallas as pl
from jax.experimental.pallas import tpu as pltpu
from jax.experimental.pallas import tpu_sc as plsc

ROUND = 64


@functools.cache
def _build(B, V, D):
    info = plsc.get_sparse_core_info()
    nw = info.num_cores * info.num_subcores
    b_per_w = B // nw
    n_rounds = b_per_w // ROUND
    mesh = plsc.VectorSubcoreMesh(core_axis_name="c", subcore_axis_name="s")

    @functools.partial(
        pl.kernel,
        mesh=mesh,
        out_type=jax.ShapeDtypeStruct((B, D), jnp.float32),
        compiler_params=pltpu.CompilerParams(use_tc_tiling_on_sc=True),
        scratch_types=[
            pltpu.VMEM((b_per_w,), jnp.int32),
            pltpu.VMEM((b_per_w, D), jnp.float32),
            pltpu.SemaphoreType.DMA,
        ],
    )
    def k(idx_hbm, table_hbm, out_hbm, idx_v, rows_v, sem):
        wid = lax.axis_index("s") * info.num_cores + lax.axis_index("c")
        base = wid * b_per_w
        pltpu.sync_copy(idx_hbm.at[pl.ds(base, b_per_w)], idx_v)

        def round_body(r, carry):
            def fire(g, carry):
                j0 = r * ROUND + g * 16
                vec = idx_v[pl.ds(j0, 16)]
                for l in range(16):
                    pltpu.async_copy(
                        table_hbm.at[pl.ds(vec[l], 1), :],
                        rows_v.at[pl.ds(j0 + l, 1), :],
                        sem,
                    )
                return carry

            lax.fori_loop(0, ROUND // 16, fire, carry, unroll=1)
            pltpu.make_async_copy(
                table_hbm.at[pl.ds(0, ROUND), :],
                rows_v.at[pl.ds(r * ROUND, ROUND), :],
                sem,
            ).wait()
            return carry

        lax.fori_loop(0, n_rounds, round_body, 0, unroll=1)
        pltpu.sync_copy(rows_v, out_hbm.at[pl.ds(base, b_per_w), :])

    return k


def kernel(nombre, table):
    B = nombre.shape[0]
    V, D = table.shape
    idx = nombre.astype(jnp.int32)
    return _build(B, V, D)(idx, table)


# speedup vs baseline: 2.2080x; 1.7124x over previous
import functools
import jax
import jax.numpy as jnp
from jax import lax
from jax.experimental import pallas as pl
from jax.experimental.pallas import tpu as pltpu
from jax.experimental.pallas import tpu_sc as plsc


@functools.cache
def _build(B, V, D):
    info = plsc.get_sparse_core_info()
    nw = info.num_cores * info.num_subcores
    assert D == nw
    half = B // 2
    mesh = plsc.VectorSubcoreMesh(core_axis_name="c", subcore_axis_name="s")

    @functools.partial(
        pl.kernel,
        mesh=mesh,
        out_type=jax.ShapeDtypeStruct((D, B), jnp.float32),
        compiler_params=pltpu.CompilerParams(
            use_tc_tiling_on_sc=True, needs_layout_passes=False
        ),
        scratch_types=[
            pltpu.VMEM((V,), jnp.float32),
            pltpu.VMEM((half,), jnp.int32),
            pltpu.VMEM((B,), jnp.float32),
            pltpu.SemaphoreType.DMA,
        ],
    )
    def k(idx_hbm, tablet_hbm, outt_hbm, row_v, idx_v, out_v, sem_row):
        d = lax.axis_index("s") * info.num_cores + lax.axis_index("c")
        rowcp = pltpu.make_async_copy(tablet_hbm.at[d], row_v, sem_row)
        rowcp.start()

        for h in range(2):
            pltpu.sync_copy(idx_hbm.at[pl.ds(h * half, half)], idx_v)
            if h == 0:
                rowcp.wait()

            def body(sg, carry):
                for l in range(8):
                    vec = idx_v[pl.ds(sg * 128 + l * 16, 16)]
                    vals = plsc.load_gather(row_v, [vec])
                    out_v[pl.ds(h * half + sg * 128 + l * 16, 16)] = vals
                return carry

            lax.fori_loop(0, half // 128, body, 0, unroll=2)

        pltpu.sync_copy(out_v, outt_hbm.at[d])

    return k


def kernel(nombre, table):
    B = nombre.shape[0]
    V, D = table.shape
    idx = nombre.astype(jnp.int32)
    outt = _build(B, V, D)(idx, table.T)
    return outt.T


# speedup vs baseline: 2.2223x; 1.0065x over previous
import functools
import jax
import jax.numpy as jnp
from jax import lax
from jax.experimental import pallas as pl
from jax.experimental.pallas import tpu as pltpu
from jax.experimental.pallas import tpu_sc as plsc

QUARTERS = 4


@functools.cache
def _build(B, V, D):
    info = plsc.get_sparse_core_info()
    nw = info.num_cores * info.num_subcores
    assert D == nw
    q = B // QUARTERS
    mesh = plsc.VectorSubcoreMesh(core_axis_name="c", subcore_axis_name="s")

    @functools.partial(
        pl.kernel,
        mesh=mesh,
        out_type=jax.ShapeDtypeStruct((D, B), jnp.float32),
        compiler_params=pltpu.CompilerParams(
            use_tc_tiling_on_sc=True, needs_layout_passes=False
        ),
        scratch_types=[
            pltpu.VMEM((V,), jnp.float32),
            pltpu.VMEM((B,), jnp.int32),
            pltpu.VMEM((2, q), jnp.float32),
            pltpu.SemaphoreType.DMA,
            pltpu.SemaphoreType.DMA,
            pltpu.SemaphoreType.DMA,
        ],
    )
    def k(idx_hbm, tablet_hbm, outt_hbm, row_v, idx_v, out_v, sem_row,
          sem_idx, sem_out):
        d = lax.axis_index("s") * info.num_cores + lax.axis_index("c")
        rowcp = pltpu.make_async_copy(tablet_hbm.at[d], row_v, sem_row)
        rowcp.start()
        idxcp = pltpu.make_async_copy(idx_hbm, idx_v, sem_idx)
        idxcp.start()
        idxcp.wait()
        rowcp.wait()

        outcps = [None, None]
        for c in range(QUARTERS):
            buf = c % 2
            if outcps[buf] is not None:
                outcps[buf].wait()

            def body(sg, carry, c=c, buf=buf):
                for l in range(8):
                    vec = idx_v[pl.ds(c * q + sg * 128 + l * 16, 16)]
                    vals = plsc.load_gather(row_v, [vec])
                    out_v[buf, pl.ds(sg * 128 + l * 16, 16)] = vals
                return carry

            lax.fori_loop(0, q // 128, body, 0, unroll=4)
            cp = pltpu.make_async_copy(
                out_v.at[buf], outt_hbm.at[d, pl.ds(c * q, q)], sem_out
            )
            cp.start()
            outcps[buf] = cp
        for cp in outcps:
            cp.wait()

    return k


def kernel(nombre, table):
    B = nombre.shape[0]
    V, D = table.shape
    idx = nombre.astype(jnp.int32)
    outt = _build(B, V, D)(idx, table.T)
    return outt.T


# speedup vs baseline: 2.6522x; 1.1934x over previous
import functools
import jax
import jax.numpy as jnp
from jax import lax
from jax.experimental import pallas as pl
from jax.experimental.pallas import tpu as pltpu
from jax.experimental.pallas import tpu_sc as plsc

QUARTERS = 4


@functools.cache
def _build(B, V, D):
    info = plsc.get_sparse_core_info()
    nw = info.num_cores * info.num_subcores
    assert D == nw
    q = B // QUARTERS
    mesh = plsc.VectorSubcoreMesh(core_axis_name="c", subcore_axis_name="s")

    @functools.partial(
        pl.kernel,
        mesh=mesh,
        out_type=jax.ShapeDtypeStruct((D, B), jnp.float32),
        compiler_params=pltpu.CompilerParams(
            use_tc_tiling_on_sc=True, needs_layout_passes=False
        ),
        scratch_types=[
            pltpu.VMEM((V,), jnp.float32),
            pltpu.VMEM((B,), jnp.int32),
            pltpu.VMEM((2, q), jnp.float32),
            pltpu.SemaphoreType.DMA,
            pltpu.SemaphoreType.DMA,
            pltpu.SemaphoreType.DMA,
        ],
    )
    def k(idx_hbm, tablet_hbm, outt_hbm, row_v, idx_v, out_v, sem_row,
          sem_idx, sem_out):
        d = lax.axis_index("s") * info.num_cores + lax.axis_index("c")
        rowcp = pltpu.make_async_copy(tablet_hbm.at[d], row_v, sem_row)
        rowcp.start()
        idxcp = pltpu.make_async_copy(idx_hbm, idx_v, sem_idx)
        idxcp.start()
        idxcp.wait()
        rowcp.wait()

        outcps = [None, None]
        for c in range(QUARTERS):
            buf = c % 2
            if outcps[buf] is not None:
                outcps[buf].wait()

            def body(sg, carry, c=c, buf=buf):
                vecs = [
                    idx_v[pl.ds(c * q + sg * 128 + l * 16, 16)]
                    for l in range(8)
                ]
                vals = [plsc.load_gather(row_v, [v]) for v in vecs]
                for l in range(8):
                    out_v[buf, pl.ds(sg * 128 + l * 16, 16)] = vals[l]
                return carry

            lax.fori_loop(0, q // 128, body, 0, unroll=4)
            cp = pltpu.make_async_copy(
                out_v.at[buf], outt_hbm.at[d, pl.ds(c * q, q)], sem_out
            )
            cp.start()
            outcps[buf] = cp
        for cp in outcps:
            cp.wait()

    return k


def kernel(nombre, table):
    B = nombre.shape[0]
    V, D = table.shape
    idx = nombre.astype(jnp.int32)
    outt = _build(B, V, D)(idx, table.T)
    return outt.T


# speedup vs baseline: 2.6744x; 1.0084x over previous
import functools
import jax
import jax.numpy as jnp
from jax import lax
from jax.experimental import pallas as pl
from jax.experimental.pallas import tpu as pltpu
from jax.experimental.pallas import tpu_sc as plsc

QUARTERS = 4


@functools.cache
def _build(B, V, D):
    info = plsc.get_sparse_core_info()
    nw = info.num_cores * info.num_subcores
    assert D == nw
    q = B // QUARTERS
    mesh = plsc.VectorSubcoreMesh(core_axis_name="c", subcore_axis_name="s")

    @functools.partial(
        pl.kernel,
        mesh=mesh,
        out_type=jax.ShapeDtypeStruct((D, B), jnp.float32),
        compiler_params=pltpu.CompilerParams(
            use_tc_tiling_on_sc=True, needs_layout_passes=False
        ),
        scratch_types=[
            pltpu.VMEM((V,), jnp.float32),
            pltpu.VMEM((B,), jnp.int32),
            pltpu.VMEM((2, q), jnp.float32),
            pltpu.SemaphoreType.DMA,
            pltpu.SemaphoreType.DMA,
            pltpu.SemaphoreType.DMA,
        ],
    )
    def k(idx_hbm, tablet_hbm, outt_hbm, row_v, idx_v, out_v, sem_row,
          sem_idx, sem_out):
        d = lax.axis_index("s") * info.num_cores + lax.axis_index("c")
        rowcp = pltpu.make_async_copy(tablet_hbm.at[d], row_v, sem_row)
        rowcp.start()
        idxcp = pltpu.make_async_copy(idx_hbm, idx_v, sem_idx)
        idxcp.start()
        idxcp.wait()
        rowcp.wait()

        outcps = [None, None]
        for c in range(QUARTERS):
            buf = c % 2
            if outcps[buf] is not None:
                outcps[buf].wait()

            def body(sg, carry, c=c, buf=buf):
                vecs = [
                    idx_v[pl.ds(c * q + sg * 128 + l * 16, 16)]
                    for l in range(8)
                ]
                vals = [plsc.load_gather(row_v, [v]) for v in vecs]
                for l in range(8):
                    out_v[buf, pl.ds(sg * 128 + l * 16, 16)] = vals[l]
                return carry

            lax.fori_loop(0, q // 128, body, 0, unroll=8)
            cp = pltpu.make_async_copy(
                out_v.at[buf], outt_hbm.at[d, pl.ds(c * q, q)], sem_out
            )
            cp.start()
            outcps[buf] = cp
        for cp in outcps:
            cp.wait()

    return k


def kernel(nombre, table):
    B = nombre.shape[0]
    V, D = table.shape
    idx = nombre.astype(jnp.int32)
    outt = _build(B, V, D)(idx, table.T)
    return outt.T
